# baseline (speedup 1.0000x reference)
import jax
import jax.numpy as jnp
from jax.experimental import pallas as pl
from jax.experimental.pallas import tpu as pltpu


_VAR_MIN = 0
_PACK = 2


def _embed_kernel(idx_ref, tab_ref, out_ref):
    idx = idx_ref[...]
    t = idx.shape[0]
    r = tab_ref.shape[0] // _PACK
    bb, s, e = out_ref.shape
    iota_r = jax.lax.broadcasted_iota(jnp.int32, (t, r), 1)
    oh0 = jnp.where(idx[:, 0:1] == iota_r, 1.0, 0.0).astype(tab_ref.dtype)
    oh1 = jnp.where(idx[:, 1:2] == iota_r, 1.0, 0.0).astype(tab_ref.dtype)
    one_hot = jnp.concatenate([oh0, oh1], axis=1)
    res = jnp.dot(
        one_hot, tab_ref[...], preferred_element_type=jnp.float32
    ).astype(out_ref.dtype)
    rows_per_b = s // _PACK
    for b in range(bb):
        blk = res[b * rows_per_b:(b + 1) * rows_per_b, :]
        out_ref[b, 0::2, :] = blk[:, :e]
        out_ref[b, 1::2, :] = blk[:, e:]


def kernel(x, table):
    B, S = x.shape
    R, E = table.shape
    assert E == 128 and R <= 128, "kernel specialized for E == 128, R <= 128"
    N = B * S

    assert S % _PACK == 0
    b_blk = 4
    while B % b_blk:
        b_blk //= 2
    tile = (b_blk * S) // _PACK
    n_blocks = B // b_blk

    idx = (x.astype(jnp.int32) - jnp.int32(_VAR_MIN)).reshape(-1)
    idx_packed = idx.reshape(tile * n_blocks, _PACK)

    tab2 = jnp.kron(jnp.eye(_PACK, dtype=table.dtype), table)

    out = pl.pallas_call(
        _embed_kernel,
        out_shape=jax.ShapeDtypeStruct((B, S, E), table.dtype),
        grid=(n_blocks,),
        in_specs=[
            pl.BlockSpec((tile, _PACK), lambda i: (i, 0)),
            pl.BlockSpec((_PACK * R, _PACK * E), lambda i: (0, 0)),
        ],
        out_specs=pl.BlockSpec((b_blk, S, E), lambda i: (i, 0, 0)),
        compiler_params=pltpu.CompilerParams(
            dimension_semantics=("parallel",)),
    )(idx_packed, tab2)
    return out


# speedup vs baseline: 1.0374x; 1.0374x over previous
import jax
import jax.numpy as jnp
from jax.experimental import pallas as pl
from jax.experimental.pallas import tpu as pltpu


_VAR_MIN = 0
_PACK = 2


def _embed_kernel(idx_ref, tab_ref, out_ref):
    t = idx_ref.shape[1]
    r = tab_ref.shape[0] // _PACK
    bb, s, e = out_ref.shape
    iota_s = jax.lax.broadcasted_iota(jnp.int32, (r, 128), 0)
    cols = []
    for g in range(t // 128):
        ev = jnp.broadcast_to(idx_ref[0:1, g * 128:(g + 1) * 128], (r, 128))
        od = jnp.broadcast_to(idx_ref[1:2, g * 128:(g + 1) * 128], (r, 128))
        top = jnp.where(ev == iota_s, 1.0, 0.0).astype(tab_ref.dtype)
        bot = jnp.where(od == iota_s, 1.0, 0.0).astype(tab_ref.dtype)
        cols.append(jnp.concatenate([top, bot], axis=0))
    oh_t = cols[0] if len(cols) == 1 else jnp.concatenate(cols, axis=1)
    res = jax.lax.dot_general(
        oh_t, tab_ref[...],
        dimension_numbers=(((0,), (0,)), ((), ())),
        preferred_element_type=jnp.float32,
    ).astype(out_ref.dtype)
    rows_per_b = s // _PACK
    for b in range(bb):
        blk = res[b * rows_per_b:(b + 1) * rows_per_b, :]
        out_ref[b, 0::2, :] = blk[:, :e]
        out_ref[b, 1::2, :] = blk[:, e:]


def kernel(x, table):
    B, S = x.shape
    R, E = table.shape
    assert E == 128 and R <= 128, "kernel specialized for E == 128, R <= 128"
    N = B * S

    assert S % _PACK == 0
    b_blk = 4
    while B % b_blk:
        b_blk //= 2
    tile = (b_blk * S) // _PACK
    n_blocks = B // b_blk

    idx = (x.astype(jnp.int32) - jnp.int32(_VAR_MIN)).reshape(-1)
    idx_t = idx.reshape(tile * n_blocks, _PACK).T

    tab2 = jnp.kron(jnp.eye(_PACK, dtype=table.dtype), table)

    out = pl.pallas_call(
        _embed_kernel,
        out_shape=jax.ShapeDtypeStruct((B, S, E), table.dtype),
        grid=(n_blocks,),
        in_specs=[
            pl.BlockSpec((_PACK, tile), lambda i: (0, i)),
            pl.BlockSpec((_PACK * R, _PACK * E), lambda i: (0, 0)),
        ],
        out_specs=pl.BlockSpec((b_blk, S, E), lambda i: (i, 0, 0)),
        compiler_params=pltpu.CompilerParams(
            dimension_semantics=("parallel",)),
    )(idx_t, tab2)
    return out


# speedup vs baseline: 3.5262x; 3.3991x over previous
import jax
import jax.numpy as jnp
from jax.experimental import pallas as pl
from jax.experimental.pallas import tpu as pltpu


_VAR_MIN = 0
_PACK = 2


def _embed_kernel(x_ref, tab_ref, out_ref):
    r = tab_ref.shape[0] // _PACK
    bb, s, e = out_ref.shape
    half = s // _PACK
    n_g = half // 128
    iota_s = jax.lax.broadcasted_iota(jnp.int32, (r, 128), 0) + _VAR_MIN
    cols = []
    for b in range(bb):
        for g in range(n_g):
            lo = x_ref[b:b + 1, g * 128:(g + 1) * 128]
            hi = x_ref[b:b + 1, half + g * 128:half + (g + 1) * 128]
            lo_b = jnp.broadcast_to(lo, (r, 128))
            hi_b = jnp.broadcast_to(hi, (r, 128))
            top = jnp.where(lo_b == iota_s, 1.0, 0.0).astype(tab_ref.dtype)
            bot = jnp.where(hi_b == iota_s, 1.0, 0.0).astype(tab_ref.dtype)
            cols.append(jnp.concatenate([top, bot], axis=0))
    oh_t = cols[0] if len(cols) == 1 else jnp.concatenate(cols, axis=1)
    res = jax.lax.dot_general(
        oh_t, tab_ref[...],
        dimension_numbers=(((0,), (0,)), ((), ())),
        preferred_element_type=jnp.float32,
    ).astype(out_ref.dtype)
    for b in range(bb):
        blk = res[b * half:(b + 1) * half, :]
        out_ref[b, :half, :] = blk[:, :e]
        out_ref[b, half:, :] = blk[:, e:]


def kernel(x, table):
    B, S = x.shape
    R, E = table.shape
    assert E == 128 and R <= 128, "kernel specialized for E == 128, R <= 128"
    assert S % (2 * 128) == 0, "kernel assumes S divisible by 256"

    assert B % 8 == 0, "kernel assumes B divisible by 8"
    b_blk = 8
    n_blocks = B // b_blk

    tab2 = jnp.kron(jnp.eye(_PACK, dtype=table.dtype), table)

    out = pl.pallas_call(
        _embed_kernel,
        out_shape=jax.ShapeDtypeStruct((B, S, E), table.dtype),
        grid=(n_blocks,),
        in_specs=[
            pl.BlockSpec((b_blk, S), lambda i: (i, 0)),
            pl.BlockSpec((_PACK * R, _PACK * E), lambda i: (0, 0)),
        ],
        out_specs=pl.BlockSpec((b_blk, S, E), lambda i: (i, 0, 0)),
        compiler_params=pltpu.CompilerParams(
            dimension_semantics=("parallel",)),
    )(x.astype(jnp.int32), tab2)
    return out


# speedup vs baseline: 3.5465x; 1.0057x over previous
import jax
import jax.numpy as jnp
from jax.experimental import pallas as pl
from jax.experimental.pallas import tpu as pltpu


_VAR_MIN = 0
_PACK = 2


def _embed_kernel(x_ref, tab_ref, out_ref):
    r = tab_ref.shape[0] // _PACK
    bb, s, e = out_ref.shape
    half = s // _PACK
    n_g = half // 128
    iota_s = jax.lax.broadcasted_iota(jnp.int32, (r, 128), 0) + _VAR_MIN
    cols = []
    for b in range(bb):
        for g in range(n_g):
            lo = x_ref[b:b + 1, g * 128:(g + 1) * 128]
            hi = x_ref[b:b + 1, half + g * 128:half + (g + 1) * 128]
            lo_b = jnp.broadcast_to(lo, (r, 128))
            hi_b = jnp.broadcast_to(hi, (r, 128))
            top = jnp.where(lo_b == iota_s, 1.0, 0.0).astype(tab_ref.dtype)
            bot = jnp.where(hi_b == iota_s, 1.0, 0.0).astype(tab_ref.dtype)
            cols.append(jnp.concatenate([top, bot], axis=0))
    oh_t = cols[0] if len(cols) == 1 else jnp.concatenate(cols, axis=1)
    res = jax.lax.dot_general(
        oh_t, tab_ref[...],
        dimension_numbers=(((0,), (0,)), ((), ())),
        preferred_element_type=jnp.float32,
    ).astype(out_ref.dtype)
    for b in range(bb):
        blk = res[b * half:(b + 1) * half, :]
        out_ref[b, :half, :] = blk[:, :e]
        out_ref[b, half:, :] = blk[:, e:]


def kernel(x, table):
    B, S = x.shape
    R, E = table.shape
    assert E == 128 and R <= 128, "kernel specialized for E == 128, R <= 128"
    assert S % (2 * 128) == 0, "kernel assumes S divisible by 256"

    assert B % 8 == 0, "kernel assumes B divisible by 8"
    b_blk = 16
    n_blocks = B // b_blk

    tab2 = jnp.kron(jnp.eye(_PACK, dtype=table.dtype), table)

    out = pl.pallas_call(
        _embed_kernel,
        out_shape=jax.ShapeDtypeStruct((B, S, E), table.dtype),
        grid=(n_blocks,),
        in_specs=[
            pl.BlockSpec((b_blk, S), lambda i: (i, 0)),
            pl.BlockSpec((_PACK * R, _PACK * E), lambda i: (0, 0)),
        ],
        out_specs=pl.BlockSpec((b_blk, S, E), lambda i: (i, 0, 0)),
        compiler_params=pltpu.CompilerParams(
            dimension_semantics=("parallel",)),
    )(x.astype(jnp.int32), tab2)
    return out
